# baseline (speedup 1.0000x reference)
import functools

import jax
import jax.numpy as jnp
import numpy as np
from jax import lax
from jax.experimental import pallas as pl
from jax.experimental.pallas import tpu as pltpu
from jax.experimental.pallas import tpu_sc as plsc

_B, _M = 4, 50
_STRIDES = (8, 16, 32, 64, 128)
_LIMIT = ((-1.0, 64.0), (64.0, 128.0), (128.0, 256.0), (256.0, 512.0),
          (512.0, 999999.0))
_SIZES = (128, 64, 32, 16, 8)
_N = sum(s * s for s in _SIZES)
_NW = 32
_S = 688
_NPAD = _NW * _S
_NV = _S // 16
_BM = _B * _M
_BIG = np.float32(99999999.0)
_NLVL = 6
_LVL_CHUNK_STARTS = (1024, 1280, 1344, 1360, 1364)


def _build_loc_tables():
    xs, ys, los, his, rads = [], [], [], [], []
    for i, s in enumerate(_SIZES):
        stride = _STRIDES[i]
        sx = np.arange(0, s * stride, stride, dtype=np.float32) + stride / 2.0
        X, Y = np.meshgrid(sx, sx, indexing='xy')
        xs.append(X.reshape(-1))
        ys.append(Y.reshape(-1))
        n = s * s
        los.append(np.full(n, _LIMIT[i][0], np.float32))
        his.append(np.full(n, _LIMIT[i][1], np.float32))
        rads.append(np.full(n, stride * 1.5, np.float32))
    pad = _NPAD - _N
    x = np.concatenate(xs + [np.zeros(pad, np.float32)])
    y = np.concatenate(ys + [np.zeros(pad, np.float32)])
    lo = np.concatenate(los + [np.full(pad, 1.0, np.float32)])
    hi = np.concatenate(his + [np.zeros(pad, np.float32)])
    rad = np.concatenate(rads + [np.zeros(pad, np.float32)])
    return x, y, lo, hi, rad


_LOC_X, _LOC_Y, _LOC_LO, _LOC_HI, _LOC_RAD = _build_loc_tables()
_COORDS = np.stack([_LOC_X[:_N], _LOC_Y[:_N]], axis=-1)

_NCHUNK = _NPAD // 16
_oldchunk = np.arange(_NCHUNK).reshape(_NW, _NV)
_oldchunk = (_oldchunk % _NV) * _NW + _oldchunk // _NV
_PERM = (_oldchunk.reshape(-1, 1) * 16 + np.arange(16)).reshape(-1)
_INV = np.empty(_NPAD, np.int64)
_INV[_PERM] = np.arange(_NPAD)
_INV_N = np.ascontiguousarray(_INV[:_N])

_LOC_X = _LOC_X[_PERM]
_LOC_Y = _LOC_Y[_PERM]
_LOC_LO = _LOC_LO[_PERM]
_LOC_HI = _LOC_HI[_PERM]
_LOC_RAD = _LOC_RAD[_PERM]


def _sqrt16(x):
    i = lax.bitcast_convert_type(x, jnp.int32)
    i = jnp.int32(0x5F3759DF) - (i >> 1)
    r = lax.bitcast_convert_type(i, jnp.float32)
    half, th = jnp.float32(0.5), jnp.float32(1.5)
    for _ in range(3):
        r = r * (th - half * x * r * r)
    return x * r


_MESH = plsc.VectorSubcoreMesh(core_axis_name="c", subcore_axis_name="s")


@functools.partial(
    pl.kernel,
    mesh=_MESH,
    out_type=[
        jax.ShapeDtypeStruct((_B * _NPAD,), jnp.int32),
        jax.ShapeDtypeStruct((_B * _NPAD,), jnp.float32),
        jax.ShapeDtypeStruct((_B * _NPAD,), jnp.float32),
        jax.ShapeDtypeStruct((_B * _NPAD,), jnp.float32),
        jax.ShapeDtypeStruct((_B * _NPAD,), jnp.float32),
        jax.ShapeDtypeStruct((_B * _NPAD,), jnp.float32),
    ],
    scratch_types=[
        pltpu.VMEM((_BM * 16,), jnp.float32),
        pltpu.VMEM((_BM * 16,), jnp.float32),
        pltpu.VMEM((_BM * 16,), jnp.float32),
        pltpu.VMEM((_BM * 16,), jnp.float32),
        pltpu.VMEM((_BM * 16,), jnp.float32),
        pltpu.VMEM((_BM * 16,), jnp.float32),
        pltpu.VMEM((_BM * 16,), jnp.int32),
        pltpu.VMEM((_BM * 16,), jnp.float32),
        pltpu.VMEM((_S,), jnp.float32),
        pltpu.VMEM((_S,), jnp.float32),
        pltpu.VMEM((_S,), jnp.float32),
        pltpu.VMEM((_S,), jnp.float32),
        pltpu.VMEM((_S,), jnp.float32),
        pltpu.VMEM((_B * _S,), jnp.int32),
        pltpu.VMEM((_B * _S,), jnp.float32),
        pltpu.VMEM((_B * _S,), jnp.float32),
        pltpu.VMEM((_B * _S,), jnp.float32),
        pltpu.VMEM((_B * _S,), jnp.float32),
        pltpu.VMEM((_B * _S,), jnp.float32),
        pltpu.SMEM((_B * _NLVL * _M,), jnp.int32),
        pltpu.SMEM((_B * _NLVL,), jnp.int32),
    ],
)
def _sc_gen_targets(bx1, by1, bx2, by2, bcl,
                    lx, ly, llo, lhi, lrad,
                    ocls, ocnt, ol, ot, orr, ob,
                    gx1, gy1, gx2, gy2, gcx, gcy, gcls, gmw,
                    vx, vy, vlo, vhi, vrad,
                    bcls, bcnt, bl, bt, br, bb,
                    lsts, lcnts):
    wid = lax.axis_index("s") * 2 + lax.axis_index("c")
    base = wid * _S

    pltpu.sync_copy(bx1, gx1)
    pltpu.sync_copy(by1, gy1)
    pltpu.sync_copy(bx2, gx2)
    pltpu.sync_copy(by2, gy2)
    pltpu.sync_copy(bcl, gcls)
    pltpu.sync_copy(lx.at[pl.ds(base, _S)], vx)
    pltpu.sync_copy(ly.at[pl.ds(base, _S)], vy)
    pltpu.sync_copy(llo.at[pl.ds(base, _S)], vlo)
    pltpu.sync_copy(lhi.at[pl.ds(base, _S)], vhi)
    pltpu.sync_copy(lrad.at[pl.ds(base, _S)], vrad)

    half = jnp.float32(0.5)

    def pre(r_, c):
        o = r_ * 16
        x1v = gx1[pl.ds(o, 16)]
        x2v = gx2[pl.ds(o, 16)]
        y1v = gy1[pl.ds(o, 16)]
        y2v = gy2[pl.ds(o, 16)]
        gcx[pl.ds(o, 16)] = (x1v + x2v) * half
        gcy[pl.ds(o, 16)] = (y1v + y2v) * half
        gmw[pl.ds(o, 16)] = jnp.maximum(x2v - x1v, y2v - y1v) * half
        return c

    lax.fori_loop(0, _BM, pre, 0)

    def _lvl_of(ch):
        lv = jnp.int32(0)
        for s_ in _LVL_CHUNK_STARTS:
            lv = lv + (ch >= s_).astype(jnp.int32)
        return lv

    lvl_first = jnp.int32(0)
    lvl_last = _lvl_of((_NV - 1) * _NW + wid)

    for sl in range(_B * _NLVL):
        lcnts[sl] = jnp.int32(0)

    for lv in range(5):
        lo_t = jnp.float32(_LIMIT[lv][0])
        hi_t = jnp.float32(_LIMIT[lv][1])
        slack = jnp.float32(_STRIDES[lv] * 1.5 + 1.0)
        margin = jnp.float32(1.0)
        needed = ((jnp.int32(lv) >= lvl_first) & (jnp.int32(lv) <= lvl_last))

        @pl.when(needed)
        def _build_lv(lv=lv, lo_t=lo_t, hi_t=hi_t, slack=slack,
                      margin=margin):
            for b in range(_B):
                loff = (b * _NLVL + lv) * _M

                def app(m, cnt, b=b, loff=loff):
                    mw = gmw[pl.ds((b * _M + m) * 16, 16)][0]
                    cond = ((mw - margin <= hi_t) & (mw + slack > lo_t))

                    @pl.when(cond)
                    def _():
                        lsts[loff + cnt] = m

                    return cnt + cond.astype(jnp.int32)

                cnt = lax.fori_loop(0, _M, app, jnp.int32(0))
                lcnts[b * _NLVL + lv] = cnt

    big = jnp.full((16,), _BIG, jnp.float32)

    def body(t, c):
        b = t // _NV
        v = t - b * _NV
        off = v * 16
        x = vx[pl.ds(off, 16)]
        y = vy[pl.ds(off, 16)]
        lo = vlo[pl.ds(off, 16)]
        hi = vhi[pl.ds(off, 16)]
        rad = vrad[pl.ds(off, 16)]
        lvl = _lvl_of(v * _NW + wid)
        slot = b * _NLVL + lvl
        loff = slot * _M
        ncand = lcnts[slot]
        rowb16 = b * _M * 16

        def inner(j, carry):
            best_area, best_l, best_t, best_r, best_b, best_c = carry
            ro = rowb16 + lsts[loff + j] * 16
            l_ = x - gx1[pl.ds(ro, 16)]
            t_ = y - gy1[pl.ds(ro, 16)]
            r_ = gx2[pl.ds(ro, 16)] - x
            b_ = gy2[pl.ds(ro, 16)] - y
            area = (l_ + r_) * (t_ + b_)
            omin = jnp.minimum(jnp.minimum(l_, t_), jnp.minimum(r_, b_))
            omax = jnp.maximum(jnp.maximum(l_, t_), jnp.maximum(r_, b_))
            cl = x - gcx[pl.ds(ro, 16)]
            ct = y - gcy[pl.ds(ro, 16)]
            cmax = jnp.maximum(jnp.abs(cl), jnp.abs(ct))
            pos = ((omin > 0.0) & (omax > lo) & (omax <= hi) & (cmax < rad))
            area_m = jnp.where(pos, area, big)
            upd = area_m < best_area
            best_area = jnp.where(upd, area_m, best_area)
            best_l = jnp.where(upd, l_, best_l)
            best_t = jnp.where(upd, t_, best_t)
            best_r = jnp.where(upd, r_, best_r)
            best_b = jnp.where(upd, b_, best_b)
            best_c = jnp.where(upd, gcls[pl.ds(ro, 16)], best_c)
            return best_area, best_l, best_t, best_r, best_b, best_c

        zero = jnp.zeros((16,), jnp.float32)
        best_area, best_l, best_t, best_r, best_b, best_c = lax.fori_loop(
            0, ncand, inner,
            (big, zero, zero, zero, zero, jnp.zeros((16,), jnp.int32)))
        anyp = best_area < big
        neg1 = jnp.float32(-1.0)
        bo = b * _S + off
        bl[pl.ds(bo, 16)] = jnp.where(anyp, best_l, neg1)
        bt[pl.ds(bo, 16)] = jnp.where(anyp, best_t, neg1)
        br[pl.ds(bo, 16)] = jnp.where(anyp, best_r, neg1)
        bb[pl.ds(bo, 16)] = jnp.where(anyp, best_b, neg1)
        bcls[pl.ds(bo, 16)] = jnp.where(anyp, best_c, 0)
        lc = jnp.maximum(best_l, 0.0)
        tc = jnp.maximum(best_t, 0.0)
        rc = jnp.maximum(best_r, 0.0)
        bc = jnp.maximum(best_b, 0.0)
        ratio = (jnp.minimum(lc, rc) * jnp.minimum(tc, bc)
                 / (jnp.maximum(lc, rc) * jnp.maximum(tc, bc) + jnp.float32(1e-10)))
        safe = jnp.where(anyp, ratio, jnp.float32(1.0))
        bcnt[pl.ds(bo, 16)] = jnp.where(anyp, _sqrt16(safe), neg1)
        return c

    lax.fori_loop(0, _B * _NV, body, 0)

    for b in range(_B):
        pltpu.sync_copy(bcls.at[pl.ds(b * _S, _S)],
                        ocls.at[pl.ds(b * _NPAD + base, _S)])
        pltpu.sync_copy(bcnt.at[pl.ds(b * _S, _S)],
                        ocnt.at[pl.ds(b * _NPAD + base, _S)])
        pltpu.sync_copy(bl.at[pl.ds(b * _S, _S)],
                        ol.at[pl.ds(b * _NPAD + base, _S)])
        pltpu.sync_copy(bt.at[pl.ds(b * _S, _S)],
                        ot.at[pl.ds(b * _NPAD + base, _S)])
        pltpu.sync_copy(br.at[pl.ds(b * _S, _S)],
                        orr.at[pl.ds(b * _NPAD + base, _S)])
        pltpu.sync_copy(bb.at[pl.ds(b * _S, _S)],
                        ob.at[pl.ds(b * _NPAD + base, _S)])


def kernel(cls_logits_0, cnt_logits_0, reg_preds_0,
           cls_logits_1, cnt_logits_1, reg_preds_1,
           cls_logits_2, cnt_logits_2, reg_preds_2,
           cls_logits_3, cnt_logits_3, reg_preds_3,
           cls_logits_4, cnt_logits_4, reg_preds_4,
           gt_boxes, classes):
    g = gt_boxes.reshape(_BM, 4)
    bx1 = jnp.broadcast_to(g[:, 0:1], (_BM, 16)).reshape(_BM * 16)
    by1 = jnp.broadcast_to(g[:, 1:2], (_BM, 16)).reshape(_BM * 16)
    bx2 = jnp.broadcast_to(g[:, 2:3], (_BM, 16)).reshape(_BM * 16)
    by2 = jnp.broadcast_to(g[:, 3:4], (_BM, 16)).reshape(_BM * 16)
    bcl = jnp.broadcast_to(classes.reshape(_BM, 1), (_BM, 16)).reshape(_BM * 16)

    ocls, ocnt, ol, ot, orr, ob = _sc_gen_targets(
        bx1, by1, bx2, by2, bcl,
        jnp.asarray(_LOC_X), jnp.asarray(_LOC_Y), jnp.asarray(_LOC_LO),
        jnp.asarray(_LOC_HI), jnp.asarray(_LOC_RAD))

    inv = jnp.asarray(_INV_N)
    cls_t = jnp.take(ocls.reshape(_B, _NPAD), inv, axis=1)[:, :, None]
    cnt_t = jnp.take(ocnt.reshape(_B, _NPAD), inv, axis=1)[:, :, None]
    reg_t = jnp.stack([jnp.take(ol.reshape(_B, _NPAD), inv, axis=1),
                       jnp.take(ot.reshape(_B, _NPAD), inv, axis=1),
                       jnp.take(orr.reshape(_B, _NPAD), inv, axis=1),
                       jnp.take(ob.reshape(_B, _NPAD), inv, axis=1)],
                      axis=-1)
    coords = jnp.asarray(_COORDS)
    return cls_t, cnt_t, reg_t, coords


# speedup vs baseline: 1.6418x; 1.6418x over previous
import functools

import jax
import jax.numpy as jnp
import numpy as np
from jax import lax
from jax.experimental import pallas as pl
from jax.experimental.pallas import tpu as pltpu
from jax.experimental.pallas import tpu_sc as plsc

_B, _M = 4, 50
_STRIDES = (8, 16, 32, 64, 128)
_LIMIT = ((-1.0, 64.0), (64.0, 128.0), (128.0, 256.0), (256.0, 512.0),
          (512.0, 999999.0))
_SIZES = (128, 64, 32, 16, 8)
_N = sum(s * s for s in _SIZES)
_NW = 32
_S = 688
_NPAD = _NW * _S
_NV = _S // 16
_BM = _B * _M
_BIG = np.float32(99999999.0)
_NLVL = 6
_LVL_CHUNK_STARTS = (1024, 1280, 1344, 1360, 1364)


def _build_loc_tables():
    xs, ys, los, his, rads = [], [], [], [], []
    for i, s in enumerate(_SIZES):
        stride = _STRIDES[i]
        sx = np.arange(0, s * stride, stride, dtype=np.float32) + stride / 2.0
        X, Y = np.meshgrid(sx, sx, indexing='xy')
        xs.append(X.reshape(-1))
        ys.append(Y.reshape(-1))
        n = s * s
        los.append(np.full(n, _LIMIT[i][0], np.float32))
        his.append(np.full(n, _LIMIT[i][1], np.float32))
        rads.append(np.full(n, stride * 1.5, np.float32))
    pad = _NPAD - _N
    x = np.concatenate(xs + [np.zeros(pad, np.float32)])
    y = np.concatenate(ys + [np.zeros(pad, np.float32)])
    lo = np.concatenate(los + [np.full(pad, 1.0, np.float32)])
    hi = np.concatenate(his + [np.zeros(pad, np.float32)])
    rad = np.concatenate(rads + [np.zeros(pad, np.float32)])
    return x, y, lo, hi, rad


_LOC_X, _LOC_Y, _LOC_LO, _LOC_HI, _LOC_RAD = _build_loc_tables()
_COORDS = np.stack([_LOC_X[:_N], _LOC_Y[:_N]], axis=-1)

_NCHUNK = _NPAD // 16
_t = np.arange(_NCHUNK)
_oldchunk = (_t % _NV) * _NW + _t // _NV
_PERM = (_oldchunk.reshape(-1, 1) * 16 + np.arange(16)).reshape(-1)

_LOC_X = _LOC_X[_PERM]
_LOC_Y = _LOC_Y[_PERM]
_LOC_LO = _LOC_LO[_PERM]
_LOC_HI = _LOC_HI[_PERM]
_LOC_RAD = _LOC_RAD[_PERM]


def _sqrt16(x):
    i = lax.bitcast_convert_type(x, jnp.int32)
    i = jnp.int32(0x5F3759DF) - (i >> 1)
    r = lax.bitcast_convert_type(i, jnp.float32)
    half, th = jnp.float32(0.5), jnp.float32(1.5)
    for _ in range(3):
        r = r * (th - half * x * r * r)
    return x * r


_MESH = plsc.VectorSubcoreMesh(core_axis_name="c", subcore_axis_name="s")


@functools.partial(
    pl.kernel,
    mesh=_MESH,
    out_type=[
        jax.ShapeDtypeStruct((_B * _NPAD,), jnp.int32),
        jax.ShapeDtypeStruct((_B * _NPAD,), jnp.float32),
        jax.ShapeDtypeStruct((_B * _NPAD,), jnp.float32),
        jax.ShapeDtypeStruct((_B * _NPAD,), jnp.float32),
        jax.ShapeDtypeStruct((_B * _NPAD,), jnp.float32),
        jax.ShapeDtypeStruct((_B * _NPAD,), jnp.float32),
    ],
    scratch_types=[
        pltpu.VMEM((_BM * 16,), jnp.float32),
        pltpu.VMEM((_BM * 16,), jnp.float32),
        pltpu.VMEM((_BM * 16,), jnp.float32),
        pltpu.VMEM((_BM * 16,), jnp.float32),
        pltpu.VMEM((_BM * 16,), jnp.float32),
        pltpu.VMEM((_BM * 16,), jnp.float32),
        pltpu.VMEM((_BM * 16,), jnp.int32),
        pltpu.VMEM((_BM * 16,), jnp.float32),
        pltpu.VMEM((_S,), jnp.float32),
        pltpu.VMEM((_S,), jnp.float32),
        pltpu.VMEM((_S,), jnp.float32),
        pltpu.VMEM((_S,), jnp.float32),
        pltpu.VMEM((_S,), jnp.float32),
        pltpu.VMEM((_B * _S,), jnp.int32),
        pltpu.VMEM((_B * _S,), jnp.float32),
        pltpu.VMEM((_B * _S,), jnp.float32),
        pltpu.VMEM((_B * _S,), jnp.float32),
        pltpu.VMEM((_B * _S,), jnp.float32),
        pltpu.VMEM((_B * _S,), jnp.float32),
        pltpu.SMEM((_B * _NLVL * _M,), jnp.int32),
        pltpu.SMEM((_B * _NLVL,), jnp.int32),
    ],
)
def _sc_gen_targets(bx1, by1, bx2, by2, bcl,
                    lx, ly, llo, lhi, lrad,
                    ocls, ocnt, ol, ot, orr, ob,
                    gx1, gy1, gx2, gy2, gcx, gcy, gcls, gmw,
                    vx, vy, vlo, vhi, vrad,
                    bcls, bcnt, bl, bt, br, bb,
                    lsts, lcnts):
    wid = lax.axis_index("s") * 2 + lax.axis_index("c")
    base = wid * _S

    pltpu.sync_copy(bx1, gx1)
    pltpu.sync_copy(by1, gy1)
    pltpu.sync_copy(bx2, gx2)
    pltpu.sync_copy(by2, gy2)
    pltpu.sync_copy(bcl, gcls)
    pltpu.sync_copy(lx.at[pl.ds(base, _S)], vx)
    pltpu.sync_copy(ly.at[pl.ds(base, _S)], vy)
    pltpu.sync_copy(llo.at[pl.ds(base, _S)], vlo)
    pltpu.sync_copy(lhi.at[pl.ds(base, _S)], vhi)
    pltpu.sync_copy(lrad.at[pl.ds(base, _S)], vrad)

    half = jnp.float32(0.5)

    def pre(r_, c):
        o = r_ * 16
        x1v = gx1[pl.ds(o, 16)]
        x2v = gx2[pl.ds(o, 16)]
        y1v = gy1[pl.ds(o, 16)]
        y2v = gy2[pl.ds(o, 16)]
        gcx[pl.ds(o, 16)] = (x1v + x2v) * half
        gcy[pl.ds(o, 16)] = (y1v + y2v) * half
        gmw[pl.ds(o, 16)] = jnp.maximum(x2v - x1v, y2v - y1v) * half
        return c

    lax.fori_loop(0, _BM, pre, 0)

    def _lvl_of(ch):
        lv = jnp.int32(0)
        for s_ in _LVL_CHUNK_STARTS:
            lv = lv + (ch >= s_).astype(jnp.int32)
        return lv

    lvl_first = jnp.int32(0)
    lvl_last = _lvl_of((_NV - 1) * _NW + wid)

    for sl in range(_B * _NLVL):
        lcnts[sl] = jnp.int32(0)

    for lv in range(5):
        lo_t = jnp.float32(_LIMIT[lv][0])
        hi_t = jnp.float32(_LIMIT[lv][1])
        slack = jnp.float32(_STRIDES[lv] * 1.5 + 1.0)
        margin = jnp.float32(1.0)
        needed = ((jnp.int32(lv) >= lvl_first) & (jnp.int32(lv) <= lvl_last))

        @pl.when(needed)
        def _build_lv(lv=lv, lo_t=lo_t, hi_t=hi_t, slack=slack,
                      margin=margin):
            for b in range(_B):
                loff = (b * _NLVL + lv) * _M

                def app(m, cnt, b=b, loff=loff):
                    mw = gmw[pl.ds((b * _M + m) * 16, 16)][0]
                    cond = ((mw - margin <= hi_t) & (mw + slack > lo_t))

                    @pl.when(cond)
                    def _():
                        lsts[loff + cnt] = m

                    return cnt + cond.astype(jnp.int32)

                cnt = lax.fori_loop(0, _M, app, jnp.int32(0))
                lcnts[b * _NLVL + lv] = cnt

    big = jnp.full((16,), _BIG, jnp.float32)

    def body(t, c):
        b = t // _NV
        v = t - b * _NV
        off = v * 16
        x = vx[pl.ds(off, 16)]
        y = vy[pl.ds(off, 16)]
        lo = vlo[pl.ds(off, 16)]
        hi = vhi[pl.ds(off, 16)]
        rad = vrad[pl.ds(off, 16)]
        lvl = _lvl_of(v * _NW + wid)
        slot = b * _NLVL + lvl
        loff = slot * _M
        ncand = lcnts[slot]
        rowb16 = b * _M * 16

        def inner(j, carry):
            best_area, best_l, best_t, best_r, best_b, best_c = carry
            ro = rowb16 + lsts[loff + j] * 16
            l_ = x - gx1[pl.ds(ro, 16)]
            t_ = y - gy1[pl.ds(ro, 16)]
            r_ = gx2[pl.ds(ro, 16)] - x
            b_ = gy2[pl.ds(ro, 16)] - y
            area = (l_ + r_) * (t_ + b_)
            omin = jnp.minimum(jnp.minimum(l_, t_), jnp.minimum(r_, b_))
            omax = jnp.maximum(jnp.maximum(l_, t_), jnp.maximum(r_, b_))
            cl = x - gcx[pl.ds(ro, 16)]
            ct = y - gcy[pl.ds(ro, 16)]
            cmax = jnp.maximum(jnp.abs(cl), jnp.abs(ct))
            pos = ((omin > 0.0) & (omax > lo) & (omax <= hi) & (cmax < rad))
            area_m = jnp.where(pos, area, big)
            upd = area_m < best_area
            best_area = jnp.where(upd, area_m, best_area)
            best_l = jnp.where(upd, l_, best_l)
            best_t = jnp.where(upd, t_, best_t)
            best_r = jnp.where(upd, r_, best_r)
            best_b = jnp.where(upd, b_, best_b)
            best_c = jnp.where(upd, gcls[pl.ds(ro, 16)], best_c)
            return best_area, best_l, best_t, best_r, best_b, best_c

        zero = jnp.zeros((16,), jnp.float32)
        best_area, best_l, best_t, best_r, best_b, best_c = lax.fori_loop(
            0, ncand, inner,
            (big, zero, zero, zero, zero, jnp.zeros((16,), jnp.int32)))
        anyp = best_area < big
        neg1 = jnp.float32(-1.0)
        bo = b * _S + off
        bl[pl.ds(bo, 16)] = jnp.where(anyp, best_l, neg1)
        bt[pl.ds(bo, 16)] = jnp.where(anyp, best_t, neg1)
        br[pl.ds(bo, 16)] = jnp.where(anyp, best_r, neg1)
        bb[pl.ds(bo, 16)] = jnp.where(anyp, best_b, neg1)
        bcls[pl.ds(bo, 16)] = jnp.where(anyp, best_c, 0)
        lc = jnp.maximum(best_l, 0.0)
        tc = jnp.maximum(best_t, 0.0)
        rc = jnp.maximum(best_r, 0.0)
        bc = jnp.maximum(best_b, 0.0)
        ratio = (jnp.minimum(lc, rc) * jnp.minimum(tc, bc)
                 / (jnp.maximum(lc, rc) * jnp.maximum(tc, bc) + jnp.float32(1e-10)))
        safe = jnp.where(anyp, ratio, jnp.float32(1.0))
        bcnt[pl.ds(bo, 16)] = jnp.where(anyp, _sqrt16(safe), neg1)
        return c

    lax.fori_loop(0, _B * _NV, body, 0)

    for b in range(_B):
        pltpu.sync_copy(bcls.at[pl.ds(b * _S, _S)],
                        ocls.at[pl.ds(b * _NPAD + base, _S)])
        pltpu.sync_copy(bcnt.at[pl.ds(b * _S, _S)],
                        ocnt.at[pl.ds(b * _NPAD + base, _S)])
        pltpu.sync_copy(bl.at[pl.ds(b * _S, _S)],
                        ol.at[pl.ds(b * _NPAD + base, _S)])
        pltpu.sync_copy(bt.at[pl.ds(b * _S, _S)],
                        ot.at[pl.ds(b * _NPAD + base, _S)])
        pltpu.sync_copy(br.at[pl.ds(b * _S, _S)],
                        orr.at[pl.ds(b * _NPAD + base, _S)])
        pltpu.sync_copy(bb.at[pl.ds(b * _S, _S)],
                        ob.at[pl.ds(b * _NPAD + base, _S)])


def kernel(cls_logits_0, cnt_logits_0, reg_preds_0,
           cls_logits_1, cnt_logits_1, reg_preds_1,
           cls_logits_2, cnt_logits_2, reg_preds_2,
           cls_logits_3, cnt_logits_3, reg_preds_3,
           cls_logits_4, cnt_logits_4, reg_preds_4,
           gt_boxes, classes):
    g = gt_boxes.reshape(_BM, 4)
    bx1 = jnp.broadcast_to(g[:, 0:1], (_BM, 16)).reshape(_BM * 16)
    by1 = jnp.broadcast_to(g[:, 1:2], (_BM, 16)).reshape(_BM * 16)
    bx2 = jnp.broadcast_to(g[:, 2:3], (_BM, 16)).reshape(_BM * 16)
    by2 = jnp.broadcast_to(g[:, 3:4], (_BM, 16)).reshape(_BM * 16)
    bcl = jnp.broadcast_to(classes.reshape(_BM, 1), (_BM, 16)).reshape(_BM * 16)

    ocls, ocnt, ol, ot, orr, ob = _sc_gen_targets(
        bx1, by1, bx2, by2, bcl,
        jnp.asarray(_LOC_X), jnp.asarray(_LOC_Y), jnp.asarray(_LOC_LO),
        jnp.asarray(_LOC_HI), jnp.asarray(_LOC_RAD))

    def _unperm(a):
        return (a.reshape(_B, _NW, _NV, 16).swapaxes(1, 2)
                .reshape(_B, _NPAD)[:, :_N])

    cls_t = _unperm(ocls)[:, :, None]
    cnt_t = _unperm(ocnt)[:, :, None]
    reg_t = jnp.stack([_unperm(ol), _unperm(ot), _unperm(orr), _unperm(ob)],
                      axis=-1)
    coords = jnp.asarray(_COORDS)
    return cls_t, cnt_t, reg_t, coords


# speedup vs baseline: 2.5594x; 1.5589x over previous
import functools

import jax
import jax.numpy as jnp
import numpy as np
from jax import lax
from jax.experimental import pallas as pl
from jax.experimental.pallas import tpu as pltpu
from jax.experimental.pallas import tpu_sc as plsc

_B, _M = 4, 50
_STRIDES = (8, 16, 32, 64, 128)
_LIMIT = ((-1.0, 64.0), (64.0, 128.0), (128.0, 256.0), (256.0, 512.0),
          (512.0, 999999.0))
_SIZES = (128, 64, 32, 16, 8)
_N = sum(s * s for s in _SIZES)
_NW = 32
_S = 688
_NPAD = _NW * _S
_NV = _S // 16
_BM = _B * _M
_BIG = np.float32(99999999.0)
_NLVL = 6
_LVL_CHUNK_STARTS = (1024, 1280, 1344, 1360, 1364)


def _build_loc_tables():
    xs, ys, los, his, rads = [], [], [], [], []
    for i, s in enumerate(_SIZES):
        stride = _STRIDES[i]
        sx = np.arange(0, s * stride, stride, dtype=np.float32) + stride / 2.0
        X, Y = np.meshgrid(sx, sx, indexing='xy')
        xs.append(X.reshape(-1))
        ys.append(Y.reshape(-1))
        n = s * s
        los.append(np.full(n, _LIMIT[i][0], np.float32))
        his.append(np.full(n, _LIMIT[i][1], np.float32))
        rads.append(np.full(n, stride * 1.5, np.float32))
    pad = _NPAD - _N
    x = np.concatenate(xs + [np.zeros(pad, np.float32)])
    y = np.concatenate(ys + [np.zeros(pad, np.float32)])
    lo = np.concatenate(los + [np.full(pad, 1.0, np.float32)])
    hi = np.concatenate(his + [np.zeros(pad, np.float32)])
    rad = np.concatenate(rads + [np.zeros(pad, np.float32)])
    return x, y, lo, hi, rad


_LOC_X, _LOC_Y, _LOC_LO, _LOC_HI, _LOC_RAD = _build_loc_tables()
_COORDS = np.stack([_LOC_X[:_N], _LOC_Y[:_N]], axis=-1)

_NCHUNK = _NPAD // 16
_t = np.arange(_NCHUNK)
_oldchunk = (_t % _NV) * _NW + _t // _NV
_PERM = (_oldchunk.reshape(-1, 1) * 16 + np.arange(16)).reshape(-1)

_LOC_X = _LOC_X[_PERM]
_LOC_Y = _LOC_Y[_PERM]
_LOC_LO = _LOC_LO[_PERM]
_LOC_HI = _LOC_HI[_PERM]
_LOC_RAD = _LOC_RAD[_PERM]


def _sqrt16(x):
    i = lax.bitcast_convert_type(x, jnp.int32)
    i = jnp.int32(0x5F3759DF) - (i >> 1)
    r = lax.bitcast_convert_type(i, jnp.float32)
    half, th = jnp.float32(0.5), jnp.float32(1.5)
    for _ in range(3):
        r = r * (th - half * x * r * r)
    return x * r


_MESH = plsc.VectorSubcoreMesh(core_axis_name="c", subcore_axis_name="s")


@functools.partial(
    pl.kernel,
    mesh=_MESH,
    out_type=[
        jax.ShapeDtypeStruct((_B * _NPAD,), jnp.int32),
        jax.ShapeDtypeStruct((_B * _NPAD,), jnp.float32),
        jax.ShapeDtypeStruct((_B * _NPAD,), jnp.float32),
        jax.ShapeDtypeStruct((_B * _NPAD,), jnp.float32),
        jax.ShapeDtypeStruct((_B * _NPAD,), jnp.float32),
        jax.ShapeDtypeStruct((_B * _NPAD,), jnp.float32),
    ],
    scratch_types=[
        pltpu.VMEM((_BM * 16,), jnp.float32),
        pltpu.VMEM((_BM * 16,), jnp.float32),
        pltpu.VMEM((_BM * 16,), jnp.float32),
        pltpu.VMEM((_BM * 16,), jnp.float32),
        pltpu.VMEM((_BM * 16,), jnp.float32),
        pltpu.VMEM((_BM * 16,), jnp.float32),
        pltpu.VMEM((_BM * 16,), jnp.int32),
        pltpu.VMEM((_BM * 16,), jnp.float32),
        pltpu.VMEM((_S,), jnp.float32),
        pltpu.VMEM((_S,), jnp.float32),
        pltpu.VMEM((_S,), jnp.float32),
        pltpu.VMEM((_S,), jnp.float32),
        pltpu.VMEM((_S,), jnp.float32),
        pltpu.VMEM((_B * _S,), jnp.int32),
        pltpu.VMEM((_B * _S,), jnp.float32),
        pltpu.VMEM((_B * _S,), jnp.float32),
        pltpu.VMEM((_B * _S,), jnp.float32),
        pltpu.VMEM((_B * _S,), jnp.float32),
        pltpu.VMEM((_B * _S,), jnp.float32),
        pltpu.SMEM((_B * _NLVL * _M,), jnp.int32),
        pltpu.SMEM((_B * _NLVL,), jnp.int32),
        pltpu.SemaphoreType.DMA,
    ],
)
def _sc_gen_targets(bx1, by1, bx2, by2, bcl,
                    lx, ly, llo, lhi, lrad,
                    ocls, ocnt, ol, ot, orr, ob,
                    gx1, gy1, gx2, gy2, gcx, gcy, gcls, gmw,
                    vx, vy, vlo, vhi, vrad,
                    bcls, bcnt, bl, bt, br, bb,
                    lsts, lcnts, osem):
    wid = lax.axis_index("s") * 2 + lax.axis_index("c")
    base = wid * _S

    pltpu.sync_copy(bx1, gx1)
    pltpu.sync_copy(by1, gy1)
    pltpu.sync_copy(bx2, gx2)
    pltpu.sync_copy(by2, gy2)
    pltpu.sync_copy(bcl, gcls)
    pltpu.sync_copy(lx.at[pl.ds(base, _S)], vx)
    pltpu.sync_copy(ly.at[pl.ds(base, _S)], vy)
    pltpu.sync_copy(llo.at[pl.ds(base, _S)], vlo)
    pltpu.sync_copy(lhi.at[pl.ds(base, _S)], vhi)
    pltpu.sync_copy(lrad.at[pl.ds(base, _S)], vrad)

    half = jnp.float32(0.5)

    def pre(r_, c):
        o = r_ * 16
        x1v = gx1[pl.ds(o, 16)]
        x2v = gx2[pl.ds(o, 16)]
        y1v = gy1[pl.ds(o, 16)]
        y2v = gy2[pl.ds(o, 16)]
        gcx[pl.ds(o, 16)] = (x1v + x2v) * half
        gcy[pl.ds(o, 16)] = (y1v + y2v) * half
        gmw[pl.ds(o, 16)] = jnp.maximum(x2v - x1v, y2v - y1v) * half
        return c

    lax.fori_loop(0, _BM, pre, 0)

    def _lvl_of(ch):
        lv = jnp.int32(0)
        for s_ in _LVL_CHUNK_STARTS:
            lv = lv + (ch >= s_).astype(jnp.int32)
        return lv

    lvl_first = jnp.int32(0)
    lvl_last = _lvl_of((_NV - 1) * _NW + wid)

    for sl in range(_B * _NLVL):
        lcnts[sl] = jnp.int32(0)

    for lv in range(5):
        lo_t = jnp.float32(_LIMIT[lv][0])
        hi_t = jnp.float32(_LIMIT[lv][1])
        slack = jnp.float32(_STRIDES[lv] * 1.5 + 1.0)
        margin = jnp.float32(1.0)
        needed = ((jnp.int32(lv) >= lvl_first) & (jnp.int32(lv) <= lvl_last))

        @pl.when(needed)
        def _build_lv(lv=lv, lo_t=lo_t, hi_t=hi_t, slack=slack,
                      margin=margin):
            for b in range(_B):
                loff = (b * _NLVL + lv) * _M

                def app(m, cnt, b=b, loff=loff):
                    mw = gmw[pl.ds((b * _M + m) * 16, 16)][0]
                    cond = ((mw - margin <= hi_t) & (mw + slack > lo_t))

                    @pl.when(cond)
                    def _():
                        lsts[loff + cnt] = m

                    return cnt + cond.astype(jnp.int32)

                cnt = lax.fori_loop(0, _M, app, jnp.int32(0))
                lcnts[b * _NLVL + lv] = cnt

    big = jnp.full((16,), _BIG, jnp.float32)

    def body(t, c):
        b = t // _NV
        v = t - b * _NV
        off = v * 16
        x = vx[pl.ds(off, 16)]
        y = vy[pl.ds(off, 16)]
        lo = vlo[pl.ds(off, 16)]
        hi = vhi[pl.ds(off, 16)]
        rad = vrad[pl.ds(off, 16)]
        lvl = _lvl_of(v * _NW + wid)
        slot = b * _NLVL + lvl
        loff = slot * _M
        ncand = lcnts[slot]
        rowb16 = b * _M * 16

        def inner(j, carry):
            best_area, best_l, best_t, best_r, best_b, best_c = carry
            ro = rowb16 + lsts[loff + j] * 16
            l_ = x - gx1[pl.ds(ro, 16)]
            t_ = y - gy1[pl.ds(ro, 16)]
            r_ = gx2[pl.ds(ro, 16)] - x
            b_ = gy2[pl.ds(ro, 16)] - y
            area = (l_ + r_) * (t_ + b_)
            omin = jnp.minimum(jnp.minimum(l_, t_), jnp.minimum(r_, b_))
            omax = jnp.maximum(jnp.maximum(l_, t_), jnp.maximum(r_, b_))
            cl = x - gcx[pl.ds(ro, 16)]
            ct = y - gcy[pl.ds(ro, 16)]
            cmax = jnp.maximum(jnp.abs(cl), jnp.abs(ct))
            pos = ((omin > 0.0) & (omax > lo) & (omax <= hi) & (cmax < rad))
            area_m = jnp.where(pos, area, big)
            upd = area_m < best_area
            best_area = jnp.where(upd, area_m, best_area)
            best_l = jnp.where(upd, l_, best_l)
            best_t = jnp.where(upd, t_, best_t)
            best_r = jnp.where(upd, r_, best_r)
            best_b = jnp.where(upd, b_, best_b)
            best_c = jnp.where(upd, gcls[pl.ds(ro, 16)], best_c)
            return best_area, best_l, best_t, best_r, best_b, best_c

        zero = jnp.zeros((16,), jnp.float32)
        best_area, best_l, best_t, best_r, best_b, best_c = lax.fori_loop(
            0, ncand, inner,
            (big, zero, zero, zero, zero, jnp.zeros((16,), jnp.int32)))
        anyp = best_area < big
        neg1 = jnp.float32(-1.0)
        bo = b * _S + off
        bl[pl.ds(bo, 16)] = jnp.where(anyp, best_l, neg1)
        bt[pl.ds(bo, 16)] = jnp.where(anyp, best_t, neg1)
        br[pl.ds(bo, 16)] = jnp.where(anyp, best_r, neg1)
        bb[pl.ds(bo, 16)] = jnp.where(anyp, best_b, neg1)
        bcls[pl.ds(bo, 16)] = jnp.where(anyp, best_c, 0)
        lc = jnp.maximum(best_l, 0.0)
        tc = jnp.maximum(best_t, 0.0)
        rc = jnp.maximum(best_r, 0.0)
        bc = jnp.maximum(best_b, 0.0)
        ratio = (jnp.minimum(lc, rc) * jnp.minimum(tc, bc)
                 / (jnp.maximum(lc, rc) * jnp.maximum(tc, bc) + jnp.float32(1e-10)))
        safe = jnp.where(anyp, ratio, jnp.float32(1.0))
        bcnt[pl.ds(bo, 16)] = jnp.where(anyp, _sqrt16(safe), neg1)
        oo = b * _NPAD + (v * _NW + wid) * 16
        pltpu.async_copy(bcls.at[pl.ds(bo, 16)], ocls.at[pl.ds(oo, 16)], osem)
        pltpu.async_copy(bcnt.at[pl.ds(bo, 16)], ocnt.at[pl.ds(oo, 16)], osem)
        pltpu.async_copy(bl.at[pl.ds(bo, 16)], ol.at[pl.ds(oo, 16)], osem)
        pltpu.async_copy(bt.at[pl.ds(bo, 16)], ot.at[pl.ds(oo, 16)], osem)
        pltpu.async_copy(br.at[pl.ds(bo, 16)], orr.at[pl.ds(oo, 16)], osem)
        pltpu.async_copy(bb.at[pl.ds(bo, 16)], ob.at[pl.ds(oo, 16)], osem)
        return c

    lax.fori_loop(0, _B * _NV, body, 0)

    def drain(t, c):
        b = t // _NV
        v = t - b * _NV
        bo = b * _S + v * 16
        oo = b * _NPAD + (v * _NW + wid) * 16
        pltpu.make_async_copy(bcls.at[pl.ds(bo, 16)],
                              ocls.at[pl.ds(oo, 16)], osem).wait()
        pltpu.make_async_copy(bcnt.at[pl.ds(bo, 16)],
                              ocnt.at[pl.ds(oo, 16)], osem).wait()
        pltpu.make_async_copy(bl.at[pl.ds(bo, 16)],
                              ol.at[pl.ds(oo, 16)], osem).wait()
        pltpu.make_async_copy(bt.at[pl.ds(bo, 16)],
                              ot.at[pl.ds(oo, 16)], osem).wait()
        pltpu.make_async_copy(br.at[pl.ds(bo, 16)],
                              orr.at[pl.ds(oo, 16)], osem).wait()
        pltpu.make_async_copy(bb.at[pl.ds(bo, 16)],
                              ob.at[pl.ds(oo, 16)], osem).wait()
        return c

    lax.fori_loop(0, _B * _NV, drain, 0)


def kernel(cls_logits_0, cnt_logits_0, reg_preds_0,
           cls_logits_1, cnt_logits_1, reg_preds_1,
           cls_logits_2, cnt_logits_2, reg_preds_2,
           cls_logits_3, cnt_logits_3, reg_preds_3,
           cls_logits_4, cnt_logits_4, reg_preds_4,
           gt_boxes, classes):
    g = gt_boxes.reshape(_BM, 4)
    bx1 = jnp.broadcast_to(g[:, 0:1], (_BM, 16)).reshape(_BM * 16)
    by1 = jnp.broadcast_to(g[:, 1:2], (_BM, 16)).reshape(_BM * 16)
    bx2 = jnp.broadcast_to(g[:, 2:3], (_BM, 16)).reshape(_BM * 16)
    by2 = jnp.broadcast_to(g[:, 3:4], (_BM, 16)).reshape(_BM * 16)
    bcl = jnp.broadcast_to(classes.reshape(_BM, 1), (_BM, 16)).reshape(_BM * 16)

    ocls, ocnt, ol, ot, orr, ob = _sc_gen_targets(
        bx1, by1, bx2, by2, bcl,
        jnp.asarray(_LOC_X), jnp.asarray(_LOC_Y), jnp.asarray(_LOC_LO),
        jnp.asarray(_LOC_HI), jnp.asarray(_LOC_RAD))

    cls_t = ocls.reshape(_B, _NPAD)[:, :_N, None]
    cnt_t = ocnt.reshape(_B, _NPAD)[:, :_N, None]
    reg_t = jnp.stack([ol.reshape(_B, _NPAD)[:, :_N],
                       ot.reshape(_B, _NPAD)[:, :_N],
                       orr.reshape(_B, _NPAD)[:, :_N],
                       ob.reshape(_B, _NPAD)[:, :_N]], axis=-1)
    coords = jnp.asarray(_COORDS)
    return cls_t, cnt_t, reg_t, coords


# speedup vs baseline: 2.6544x; 1.0371x over previous
import functools

import jax
import jax.numpy as jnp
import numpy as np
from jax import lax
from jax.experimental import pallas as pl
from jax.experimental.pallas import tpu as pltpu
from jax.experimental.pallas import tpu_sc as plsc

_B, _M = 4, 50
_STRIDES = (8, 16, 32, 64, 128)
_LIMIT = ((-1.0, 64.0), (64.0, 128.0), (128.0, 256.0), (256.0, 512.0),
          (512.0, 999999.0))
_SIZES = (128, 64, 32, 16, 8)
_N = sum(s * s for s in _SIZES)
_NW = 32
_S = 688
_NPAD = _NW * _S
_NV = _S // 16
_BM = _B * _M
_BIG = np.float32(99999999.0)
_NLVL = 6
_LVL_CHUNK_STARTS = (1024, 1280, 1344, 1360, 1364)


def _build_loc_tables():
    xs, ys, los, his, rads = [], [], [], [], []
    for i, s in enumerate(_SIZES):
        stride = _STRIDES[i]
        sx = np.arange(0, s * stride, stride, dtype=np.float32) + stride / 2.0
        X, Y = np.meshgrid(sx, sx, indexing='xy')
        xs.append(X.reshape(-1))
        ys.append(Y.reshape(-1))
        n = s * s
        los.append(np.full(n, _LIMIT[i][0], np.float32))
        his.append(np.full(n, _LIMIT[i][1], np.float32))
        rads.append(np.full(n, stride * 1.5, np.float32))
    pad = _NPAD - _N
    x = np.concatenate(xs + [np.zeros(pad, np.float32)])
    y = np.concatenate(ys + [np.zeros(pad, np.float32)])
    lo = np.concatenate(los + [np.full(pad, 1.0, np.float32)])
    hi = np.concatenate(his + [np.zeros(pad, np.float32)])
    rad = np.concatenate(rads + [np.zeros(pad, np.float32)])
    return x, y, lo, hi, rad


_LOC_X, _LOC_Y, _LOC_LO, _LOC_HI, _LOC_RAD = _build_loc_tables()
_COORDS = np.stack([_LOC_X[:_N], _LOC_Y[:_N]], axis=-1)

_NCHUNK = _NPAD // 16
_t = np.arange(_NCHUNK)
_oldchunk = (_t % _NV) * _NW + _t // _NV
_PERM = (_oldchunk.reshape(-1, 1) * 16 + np.arange(16)).reshape(-1)

_LOC_X = _LOC_X[_PERM]
_LOC_Y = _LOC_Y[_PERM]
_LOC_LO = _LOC_LO[_PERM]
_LOC_HI = _LOC_HI[_PERM]
_LOC_RAD = _LOC_RAD[_PERM]


def _sqrt16(x):
    i = lax.bitcast_convert_type(x, jnp.int32)
    i = jnp.int32(0x5F3759DF) - (i >> 1)
    r = lax.bitcast_convert_type(i, jnp.float32)
    half, th = jnp.float32(0.5), jnp.float32(1.5)
    for _ in range(3):
        r = r * (th - half * x * r * r)
    return x * r


_MESH = plsc.VectorSubcoreMesh(core_axis_name="c", subcore_axis_name="s")


@functools.partial(
    pl.kernel,
    mesh=_MESH,
    out_type=[
        jax.ShapeDtypeStruct((_B * _NPAD,), jnp.int32),
        jax.ShapeDtypeStruct((_B * _NPAD,), jnp.float32),
        jax.ShapeDtypeStruct((_B * _NPAD,), jnp.float32),
        jax.ShapeDtypeStruct((_B * _NPAD,), jnp.float32),
        jax.ShapeDtypeStruct((_B * _NPAD,), jnp.float32),
        jax.ShapeDtypeStruct((_B * _NPAD,), jnp.float32),
    ],
    scratch_types=[
        pltpu.VMEM((_BM * 4 + 16,), jnp.float32),
        pltpu.VMEM((_BM + 16,), jnp.int32),
        pltpu.VMEM((_BM * 16,), jnp.float32),
        pltpu.VMEM((_BM * 16,), jnp.float32),
        pltpu.VMEM((_BM * 16,), jnp.float32),
        pltpu.VMEM((_BM * 16,), jnp.float32),
        pltpu.VMEM((_BM * 16,), jnp.float32),
        pltpu.VMEM((_BM * 16,), jnp.float32),
        pltpu.VMEM((_BM * 16,), jnp.int32),
        pltpu.VMEM((_BM * 16,), jnp.float32),
        pltpu.VMEM((_S,), jnp.float32),
        pltpu.VMEM((_S,), jnp.float32),
        pltpu.VMEM((_S,), jnp.float32),
        pltpu.VMEM((_S,), jnp.float32),
        pltpu.VMEM((_S,), jnp.float32),
        pltpu.VMEM((_B * _S,), jnp.int32),
        pltpu.VMEM((_B * _S,), jnp.float32),
        pltpu.VMEM((_B * _S,), jnp.float32),
        pltpu.VMEM((_B * _S,), jnp.float32),
        pltpu.VMEM((_B * _S,), jnp.float32),
        pltpu.VMEM((_B * _S,), jnp.float32),
        pltpu.SMEM((_B * _NLVL * _M,), jnp.int32),
        pltpu.SMEM((_B * _NLVL,), jnp.int32),
        pltpu.SemaphoreType.DMA,
        pltpu.SemaphoreType.DMA,
    ],
)
def _sc_gen_targets(g4, cls_in,
                    lx, ly, llo, lhi, lrad,
                    ocls, ocnt, ol, ot, orr, ob,
                    g4v, clsv,
                    gx1, gy1, gx2, gy2, gcx, gcy, gcls, gmw,
                    vx, vy, vlo, vhi, vrad,
                    bcls, bcnt, bl, bt, br, bb,
                    lsts, lcnts, isem, osem):
    wid = lax.axis_index("s") * 2 + lax.axis_index("c")
    base = wid * _S

    stage = [
        (g4, g4v.at[pl.ds(0, _BM * 4)]),
        (cls_in, clsv.at[pl.ds(0, _BM)]),
        (lx.at[pl.ds(base, _S)], vx),
        (ly.at[pl.ds(base, _S)], vy),
        (llo.at[pl.ds(base, _S)], vlo),
        (lhi.at[pl.ds(base, _S)], vhi),
        (lrad.at[pl.ds(base, _S)], vrad),
    ]
    for s_, d_ in stage:
        pltpu.async_copy(s_, d_, isem)
    for s_, d_ in stage:
        pltpu.make_async_copy(s_, d_, isem).wait()

    half = jnp.float32(0.5)

    def pre(r_, c):
        o = r_ * 16
        vals = g4v[pl.ds(r_ * 4, 16)]
        x1 = vals[0]
        y1 = vals[1]
        x2 = vals[2]
        y2 = vals[3]
        cv = clsv[pl.ds(r_, 16)][0]
        gx1[pl.ds(o, 16)] = jnp.broadcast_to(x1, (16,))
        gy1[pl.ds(o, 16)] = jnp.broadcast_to(y1, (16,))
        gx2[pl.ds(o, 16)] = jnp.broadcast_to(x2, (16,))
        gy2[pl.ds(o, 16)] = jnp.broadcast_to(y2, (16,))
        gcls[pl.ds(o, 16)] = jnp.broadcast_to(cv, (16,))
        gcx[pl.ds(o, 16)] = jnp.broadcast_to((x1 + x2) * half, (16,))
        gcy[pl.ds(o, 16)] = jnp.broadcast_to((y1 + y2) * half, (16,))
        gmw[pl.ds(o, 16)] = jnp.broadcast_to(
            jnp.maximum(x2 - x1, y2 - y1) * half, (16,))
        return c

    lax.fori_loop(0, _BM, pre, 0)

    def _lvl_of(ch):
        lv = jnp.int32(0)
        for s_ in _LVL_CHUNK_STARTS:
            lv = lv + (ch >= s_).astype(jnp.int32)
        return lv

    lvl_first = jnp.int32(0)
    lvl_last = _lvl_of((_NV - 1) * _NW + wid)

    for sl in range(_B * _NLVL):
        lcnts[sl] = jnp.int32(0)

    for lv in range(5):
        lo_t = jnp.float32(_LIMIT[lv][0])
        hi_t = jnp.float32(_LIMIT[lv][1])
        slack = jnp.float32(_STRIDES[lv] * 1.5 + 1.0)
        margin = jnp.float32(1.0)
        needed = ((jnp.int32(lv) >= lvl_first) & (jnp.int32(lv) <= lvl_last))

        @pl.when(needed)
        def _build_lv(lv=lv, lo_t=lo_t, hi_t=hi_t, slack=slack,
                      margin=margin):
            for b in range(_B):
                loff = (b * _NLVL + lv) * _M

                def app(m, cnt, b=b, loff=loff):
                    mw = gmw[pl.ds((b * _M + m) * 16, 16)][0]
                    cond = ((mw - margin <= hi_t) & (mw + slack > lo_t))

                    @pl.when(cond)
                    def _():
                        lsts[loff + cnt] = m

                    return cnt + cond.astype(jnp.int32)

                cnt = lax.fori_loop(0, _M, app, jnp.int32(0))
                lcnts[b * _NLVL + lv] = cnt

    big = jnp.full((16,), _BIG, jnp.float32)

    def body(t, c):
        b = t // _NV
        v = t - b * _NV
        off = v * 16
        x = vx[pl.ds(off, 16)]
        y = vy[pl.ds(off, 16)]
        lo = vlo[pl.ds(off, 16)]
        hi = vhi[pl.ds(off, 16)]
        rad = vrad[pl.ds(off, 16)]
        lvl = _lvl_of(v * _NW + wid)
        slot = b * _NLVL + lvl
        loff = slot * _M
        ncand = lcnts[slot]
        rowb16 = b * _M * 16

        def inner(j, carry):
            best_area, best_l, best_t, best_r, best_b, best_c = carry
            ro = rowb16 + lsts[loff + j] * 16
            l_ = x - gx1[pl.ds(ro, 16)]
            t_ = y - gy1[pl.ds(ro, 16)]
            r_ = gx2[pl.ds(ro, 16)] - x
            b_ = gy2[pl.ds(ro, 16)] - y
            area = (l_ + r_) * (t_ + b_)
            omin = jnp.minimum(jnp.minimum(l_, t_), jnp.minimum(r_, b_))
            omax = jnp.maximum(jnp.maximum(l_, t_), jnp.maximum(r_, b_))
            cl = x - gcx[pl.ds(ro, 16)]
            ct = y - gcy[pl.ds(ro, 16)]
            cmax = jnp.maximum(jnp.abs(cl), jnp.abs(ct))
            pos = ((omin > 0.0) & (omax > lo) & (omax <= hi) & (cmax < rad))
            area_m = jnp.where(pos, area, big)
            upd = area_m < best_area
            best_area = jnp.where(upd, area_m, best_area)
            best_l = jnp.where(upd, l_, best_l)
            best_t = jnp.where(upd, t_, best_t)
            best_r = jnp.where(upd, r_, best_r)
            best_b = jnp.where(upd, b_, best_b)
            best_c = jnp.where(upd, gcls[pl.ds(ro, 16)], best_c)
            return best_area, best_l, best_t, best_r, best_b, best_c

        zero = jnp.zeros((16,), jnp.float32)
        best_area, best_l, best_t, best_r, best_b, best_c = lax.fori_loop(
            0, ncand, inner,
            (big, zero, zero, zero, zero, jnp.zeros((16,), jnp.int32)))
        anyp = best_area < big
        neg1 = jnp.float32(-1.0)
        bo = b * _S + off
        bl[pl.ds(bo, 16)] = jnp.where(anyp, best_l, neg1)
        bt[pl.ds(bo, 16)] = jnp.where(anyp, best_t, neg1)
        br[pl.ds(bo, 16)] = jnp.where(anyp, best_r, neg1)
        bb[pl.ds(bo, 16)] = jnp.where(anyp, best_b, neg1)
        bcls[pl.ds(bo, 16)] = jnp.where(anyp, best_c, 0)
        lc = jnp.maximum(best_l, 0.0)
        tc = jnp.maximum(best_t, 0.0)
        rc = jnp.maximum(best_r, 0.0)
        bc = jnp.maximum(best_b, 0.0)
        ratio = (jnp.minimum(lc, rc) * jnp.minimum(tc, bc)
                 / (jnp.maximum(lc, rc) * jnp.maximum(tc, bc) + jnp.float32(1e-10)))
        safe = jnp.where(anyp, ratio, jnp.float32(1.0))
        bcnt[pl.ds(bo, 16)] = jnp.where(anyp, _sqrt16(safe), neg1)
        oo = b * _NPAD + (v * _NW + wid) * 16
        pltpu.async_copy(bcls.at[pl.ds(bo, 16)], ocls.at[pl.ds(oo, 16)], osem)
        pltpu.async_copy(bcnt.at[pl.ds(bo, 16)], ocnt.at[pl.ds(oo, 16)], osem)
        pltpu.async_copy(bl.at[pl.ds(bo, 16)], ol.at[pl.ds(oo, 16)], osem)
        pltpu.async_copy(bt.at[pl.ds(bo, 16)], ot.at[pl.ds(oo, 16)], osem)
        pltpu.async_copy(br.at[pl.ds(bo, 16)], orr.at[pl.ds(oo, 16)], osem)
        pltpu.async_copy(bb.at[pl.ds(bo, 16)], ob.at[pl.ds(oo, 16)], osem)
        return c

    lax.fori_loop(0, _B * _NV, body, 0)

    def drain(t, c):
        b = t // _NV
        v = t - b * _NV
        bo = b * _S + v * 16
        oo = b * _NPAD + (v * _NW + wid) * 16
        pltpu.make_async_copy(bcls.at[pl.ds(bo, 16)],
                              ocls.at[pl.ds(oo, 16)], osem).wait()
        pltpu.make_async_copy(bcnt.at[pl.ds(bo, 16)],
                              ocnt.at[pl.ds(oo, 16)], osem).wait()
        pltpu.make_async_copy(bl.at[pl.ds(bo, 16)],
                              ol.at[pl.ds(oo, 16)], osem).wait()
        pltpu.make_async_copy(bt.at[pl.ds(bo, 16)],
                              ot.at[pl.ds(oo, 16)], osem).wait()
        pltpu.make_async_copy(br.at[pl.ds(bo, 16)],
                              orr.at[pl.ds(oo, 16)], osem).wait()
        pltpu.make_async_copy(bb.at[pl.ds(bo, 16)],
                              ob.at[pl.ds(oo, 16)], osem).wait()
        return c

    lax.fori_loop(0, _B * _NV, drain, 0)


def kernel(cls_logits_0, cnt_logits_0, reg_preds_0,
           cls_logits_1, cnt_logits_1, reg_preds_1,
           cls_logits_2, cnt_logits_2, reg_preds_2,
           cls_logits_3, cnt_logits_3, reg_preds_3,
           cls_logits_4, cnt_logits_4, reg_preds_4,
           gt_boxes, classes):
    ocls, ocnt, ol, ot, orr, ob = _sc_gen_targets(
        gt_boxes.reshape(_BM * 4), classes.reshape(_BM),
        jnp.asarray(_LOC_X), jnp.asarray(_LOC_Y), jnp.asarray(_LOC_LO),
        jnp.asarray(_LOC_HI), jnp.asarray(_LOC_RAD))

    cls_t = ocls.reshape(_B, _NPAD)[:, :_N, None]
    cnt_t = ocnt.reshape(_B, _NPAD)[:, :_N, None]
    reg_t = jnp.stack([ol.reshape(_B, _NPAD)[:, :_N],
                       ot.reshape(_B, _NPAD)[:, :_N],
                       orr.reshape(_B, _NPAD)[:, :_N],
                       ob.reshape(_B, _NPAD)[:, :_N]], axis=-1)
    coords = jnp.asarray(_COORDS)
    return cls_t, cnt_t, reg_t, coords
